# baseline (speedup 1.0000x reference)
import functools

import jax
import jax.numpy as jnp
from jax import lax
from jax.experimental import pallas as pl
from jax.experimental.pallas import tpu as pltpu
from jax.experimental.pallas import tpu_sc as plsc

_B, _R, _D_IN, _D_OUT = 4096, 200, 128, 64
_ROWS = _B * _R
_NC, _NS = 2, 16
_NW = _NC * _NS
_RPW = _ROWS // _NW
_CHUNK = 512
_NCH = _RPW // _CHUNK


def _sc_body(x_hbm, idx_hbm, out_hbm, idx_v, in_v, out_v):
    wid = lax.axis_index("s") * _NC + lax.axis_index("c")
    pltpu.sync_copy(idx_hbm, idx_v)
    idxv = [idx_v[pl.ds(16 * k, 16)] for k in range(_D_OUT // 16)]

    def chunk_body(c, carry):
        row0 = (wid * _RPW + c * _CHUNK)
        pltpu.sync_copy(x_hbm.at[pl.ds(row0 * _D_IN, _CHUNK * _D_IN)], in_v)

        def row_body(r, carry2):
            rb = r * _D_IN
            ob = r * _D_OUT
            for k in range(_D_OUT // 16):
                v = plsc.load_gather(in_v, [idxv[k] + rb])
                out_v[pl.ds(ob + 16 * k, 16)] = v
            return carry2

        lax.fori_loop(0, _CHUNK, row_body, 0)
        pltpu.sync_copy(out_v, out_hbm.at[pl.ds(row0 * _D_OUT, _CHUNK * _D_OUT)])
        return carry

    lax.fori_loop(0, _NCH, chunk_body, 0)


def kernel(x, indices):
    x_flat = x.reshape(_ROWS * _D_IN)
    sc_call = pl.kernel(
        _sc_body,
        out_type=jax.ShapeDtypeStruct((_ROWS * _D_OUT,), jnp.float32),
        mesh=plsc.VectorSubcoreMesh(core_axis_name="c", subcore_axis_name="s"),
        scratch_types=[
            pltpu.VMEM((_D_OUT,), jnp.int32),
            pltpu.VMEM((_CHUNK * _D_IN,), jnp.float32),
            pltpu.VMEM((_CHUNK * _D_OUT,), jnp.float32),
        ],
        compiler_params=pltpu.CompilerParams(needs_layout_passes=False),
    )
    out = sc_call(x_flat, indices)
    return out.reshape(_B, _R, _D_OUT)


# speedup vs baseline: 1.6107x; 1.6107x over previous
import jax
import jax.numpy as jnp
from jax import lax
from jax.experimental import pallas as pl
from jax.experimental.pallas import tpu as pltpu
from jax.experimental.pallas import tpu_sc as plsc

_B, _R, _D_IN, _D_OUT = 4096, 200, 128, 64
_ROWS = _B * _R
_NC, _NS = 2, 16
_NW = _NC * _NS
_RPW = _ROWS // _NW
_CHUNK = 256
_NCH = _RPW // _CHUNK
_IN_W = _CHUNK * _D_IN
_OUT_W = _CHUNK * _D_OUT


def _sc_body(x_hbm, idx_hbm, out_hbm, idx_v, in_bufs, out_bufs, in_sems, out_sems):
    wid = lax.axis_index("s") * _NC + lax.axis_index("c")
    base = wid * _RPW
    pltpu.sync_copy(idx_hbm, idx_v)
    idxv = [idx_v[pl.ds(16 * k, 16)] for k in range(_D_OUT // 16)]

    def in_src(c):
        return x_hbm.at[pl.ds((base + c * _CHUNK) * _D_IN, _IN_W)]

    def out_dst(c):
        return out_hbm.at[pl.ds((base + c * _CHUNK) * _D_OUT, _OUT_W)]

    def compute(in_buf, out_buf):
        @plsc.parallel_loop(0, _CHUNK, unroll=8)
        def _(r):
            rb = r * _D_IN
            ob = r * _D_OUT
            for k in range(_D_OUT // 16):
                v = plsc.load_gather(in_buf, [idxv[k] + rb])
                out_buf[pl.ds(ob + 16 * k, 16)] = v

    pltpu.async_copy(in_src(0), in_bufs[0], in_sems[0])

    def pair_body(i, carry):
        for b in range(2):
            c = 2 * i + b
            nxt = c + 1
            @pl.when(nxt < _NCH)
            def _():
                pltpu.async_copy(in_src(nxt), in_bufs[1 - b], in_sems[1 - b])
            pltpu.make_async_copy(in_src(c), in_bufs[b], in_sems[b]).wait()
            @pl.when(c >= 2)
            def _():
                pltpu.make_async_copy(out_bufs[b], out_dst(c - 2), out_sems[b]).wait()
            compute(in_bufs[b], out_bufs[b])
            pltpu.async_copy(out_bufs[b], out_dst(c), out_sems[b])
        return carry

    lax.fori_loop(0, _NCH // 2, pair_body, 0)
    pltpu.make_async_copy(out_bufs[0], out_dst(_NCH - 2), out_sems[0]).wait()
    pltpu.make_async_copy(out_bufs[1], out_dst(_NCH - 1), out_sems[1]).wait()


def kernel(x, indices):
    x_flat = x.reshape(_ROWS * _D_IN)
    sc_call = pl.kernel(
        _sc_body,
        out_type=jax.ShapeDtypeStruct((_ROWS * _D_OUT,), jnp.float32),
        mesh=plsc.VectorSubcoreMesh(core_axis_name="c", subcore_axis_name="s"),
        scratch_types=[
            pltpu.VMEM((_D_OUT,), jnp.int32),
            [pltpu.VMEM((_IN_W,), jnp.float32) for _ in range(2)],
            [pltpu.VMEM((_OUT_W,), jnp.float32) for _ in range(2)],
            [pltpu.SemaphoreType.DMA for _ in range(2)],
            [pltpu.SemaphoreType.DMA for _ in range(2)],
        ],
        compiler_params=pltpu.CompilerParams(needs_layout_passes=False),
    )
    out = sc_call(x_flat, indices)
    return out.reshape(_B, _R, _D_OUT)


# speedup vs baseline: 1.9632x; 1.2189x over previous
import jax
import jax.numpy as jnp
from jax import lax
from jax.experimental import pallas as pl
from jax.experimental.pallas import tpu as pltpu
from jax.experimental.pallas import tpu_sc as plsc

_B, _R, _D_IN, _D_OUT = 4096, 200, 128, 64
_NC, _NS = 2, 16
_NW = _NC * _NS
_BPW = _B // _NW
_NG = _D_OUT // 16


def _sc_body(x_hbm, idx_hbm, out_hbm, idx_v, in_bufs, out_bufs, in_sems, out_sems):
    wid = lax.axis_index("s") * _NC + lax.axis_index("c")
    base = wid * _BPW
    pltpu.sync_copy(idx_hbm, idx_v)
    idxv = [idx_v[pl.ds(16 * k, 16)] for k in range(_NG)]

    def compute(in_buf, out_buf):
        @plsc.parallel_loop(0, _R, unroll=8)
        def _(r):
            row = jnp.full((16,), r, jnp.int32)
            for k in range(_NG):
                v = plsc.load_gather(in_buf, [row, idxv[k]])
                out_buf[r, pl.ds(16 * k, 16)] = v

    pltpu.async_copy(x_hbm.at[base], in_bufs[0], in_sems[0])

    def pair_body(i, carry):
        for b in range(2):
            c = 2 * i + b
            nxt = c + 1
            @pl.when(nxt < _BPW)
            def _():
                pltpu.async_copy(x_hbm.at[base + nxt], in_bufs[1 - b], in_sems[1 - b])
            pltpu.make_async_copy(x_hbm.at[base + c], in_bufs[b], in_sems[b]).wait()
            @pl.when(c >= 2)
            def _():
                pltpu.make_async_copy(
                    out_bufs[b], out_hbm.at[base + c - 2], out_sems[b]).wait()
            compute(in_bufs[b], out_bufs[b])
            pltpu.async_copy(out_bufs[b], out_hbm.at[base + c], out_sems[b])
        return carry

    lax.fori_loop(0, _BPW // 2, pair_body, 0)
    pltpu.make_async_copy(out_bufs[0], out_hbm.at[base + _BPW - 2], out_sems[0]).wait()
    pltpu.make_async_copy(out_bufs[1], out_hbm.at[base + _BPW - 1], out_sems[1]).wait()


def kernel(x, indices):
    sc_call = pl.kernel(
        _sc_body,
        out_type=jax.ShapeDtypeStruct((_B, _R, _D_OUT), jnp.float32),
        mesh=plsc.VectorSubcoreMesh(core_axis_name="c", subcore_axis_name="s"),
        scratch_types=[
            pltpu.VMEM((_D_OUT,), jnp.int32),
            [pltpu.VMEM((_R, _D_IN), jnp.float32) for _ in range(2)],
            [pltpu.VMEM((_R, _D_OUT), jnp.float32) for _ in range(2)],
            [pltpu.SemaphoreType.DMA for _ in range(2)],
            [pltpu.SemaphoreType.DMA for _ in range(2)],
        ],
        compiler_params=pltpu.CompilerParams(needs_layout_passes=False),
    )
    return sc_call(x, indices)
